# baseline (speedup 1.0000x reference)
import functools

import jax
import jax.numpy as jnp
from jax import lax
from jax.experimental import pallas as pl
from jax.experimental.pallas import tpu as pltpu
from jax.experimental.pallas import tpu_sc as plsc

B = 16384
D = 64
N = 1000000
SR = 120
STAGE = 128
NB = B // STAGE


@functools.cache
def _build_kernel():
    info = plsc.get_sparse_core_info()
    nw = info.num_cores * info.num_subcores
    per = N // nw
    mesh = plsc.VectorSubcoreMesh(core_axis_name="c", subcore_axis_name="s")
    i32 = jnp.int32

    @functools.partial(
        pl.kernel,
        mesh=mesh,
        out_type=(
            jax.ShapeDtypeStruct((B, 2 * D), jnp.float32),
            jax.ShapeDtypeStruct((B, 2 * D), jnp.float32),
        ),
        scratch_types=[
            pltpu.VMEM((B,), i32),
            pltpu.VMEM((B + 16,), i32),
            pltpu.VMEM((B + 16,), i32),
            pltpu.VMEM((SR, D), jnp.float32),
            pltpu.VMEM((SR, D), jnp.float32),
            pltpu.VMEM((STAGE, 2 * D), jnp.float32),
            pltpu.VMEM((STAGE, 2 * D), jnp.float32),
            pltpu.VMEM((NB, STAGE), i32),
            pltpu.VMEM((16,), i32),
            pltpu.VMEM((16,), i32),
            pltpu.SemaphoreType.DMA,
            pltpu.SemaphoreType.DMA,
        ],
        compiler_params=pltpu.CompilerParams(needs_layout_passes=False),
    )
    def k(ids_hbm, ws_hbm, wa_hbm, out_s_hbm, out_a_hbm,
          idx_v, sel_id, sel_pos, slab_s, slab_a, stage_s, stage_a,
          poslist, hit_id, hit_pos, sem_s, sem_a):
        wid = lax.axis_index("s") * info.num_cores + lax.axis_index("c")
        lo = wid * per // 8 * 8
        hi = (wid + 1) * per // 8 * 8
        iota = lax.iota(i32, 16)
        lane0 = iota == 0

        pltpu.sync_copy(ids_hbm, idx_v)

        def sel_body(g, off):
            v = idx_v[pl.ds(g * 16, 16)]
            m = (v >= lo) & (v < hi)
            plsc.store_compressed(sel_id.at[pl.ds(off, 16)], v, mask=m)
            plsc.store_compressed(sel_pos.at[pl.ds(off, 16)], iota + g * 16, mask=m)
            return off + plsc.all_reduce_population_count(m)[0]

        nsel = lax.fori_loop(0, B // 16, sel_body, jnp.int32(0))

        def flush(bb):
            cp_s = pltpu.async_copy(
                stage_s, out_s_hbm.at[poslist.at[bb]], sem_s)
            cp_a = pltpu.async_copy(
                stage_a, out_a_hbm.at[poslist.at[bb]], sem_a)
            cp_s.wait()
            cp_a.wait()

        n_sub = (hi - lo + SR - 1) // SR

        def sub_body(s, wrbb):
            wr, bb = wrbb
            sub_lo = lo + s * SR
            sub_hi = jnp.minimum(sub_lo + SR, hi)
            start = jnp.minimum(sub_lo, N - SR)
            pltpu.sync_copy(ws_hbm.at[pl.ds(start, SR)], slab_s)
            pltpu.sync_copy(wa_hbm.at[pl.ds(start, SR)], slab_a)

            def grp_body(g, wrbb):
                wr, bb = wrbb
                v = sel_id[pl.ds(g * 16, 16)]
                valid = (g * 16 + iota) < nsel
                m = valid & (v >= sub_lo) & (v < sub_hi)
                c = plsc.all_reduce_population_count(m)[0]
                plsc.store_compressed(hit_id.at[pl.ds(0, 16)], v, mask=m)
                plsc.store_compressed(
                    hit_pos.at[pl.ds(0, 16)], sel_pos[pl.ds(g * 16, 16)], mask=m)

                def one_hit(h, wrbb):
                    wr, bb = wrbb
                    hsp = jnp.full((16,), 0, i32) + h
                    rr = plsc.load_gather(hit_id, [hsp])[0] - start
                    pv = plsc.load_gather(hit_pos, [hsp])
                    for q in range(D // 16):
                        cs = pl.ds(q * 16, 16)
                        stage_s[wr, cs] = slab_s[rr, cs]
                        stage_a[wr, cs] = slab_a[rr, cs]
                    plsc.store_scatter(
                        poslist,
                        [jnp.full((16,), 0, i32) + bb,
                         jnp.full((16,), 0, i32) + wr],
                        pv, mask=lane0)

                    @pl.when(wr + 1 == STAGE)
                    def _():
                        flush(bb)

                    full = wr + 1 == STAGE
                    return (jnp.where(full, 0, wr + 1).astype(i32),
                            jnp.where(full, bb + 1, bb).astype(i32))

                return lax.fori_loop(0, c, one_hit, (wr, bb))

            return lax.fori_loop(0, (nsel + 15) // 16, grp_body, (wr, bb))

        wr, bb = lax.fori_loop(0, n_sub, sub_body,
                               (jnp.int32(0), jnp.int32(0)))

        @pl.when(wr > 0)
        def _():
            lastv = jnp.full((16,), 0, i32) + (wr - 1)
            bsp = jnp.full((16,), 0, i32) + bb
            pv = plsc.load_gather(poslist, [bsp, lastv])

            def pad_body(p, _):
                for q in range(D // 16):
                    cs = pl.ds(q * 16, 16)
                    stage_s[p, cs] = stage_s[wr - 1, cs]
                    stage_a[p, cs] = stage_a[wr - 1, cs]
                plsc.store_scatter(
                    poslist, [bsp, jnp.full((16,), 0, i32) + p], pv, mask=lane0)
                return 0

            lax.fori_loop(wr, STAGE, pad_body, 0)
            flush(bb)

    return k


def kernel(instance_ids, W_shape, W_appearance):
    ids = instance_ids.astype(jnp.int32)
    out_s, out_a = _build_kernel()(ids, W_shape, W_appearance)
    return (out_s[:, :D], out_a[:, :D])


# speedup vs baseline: 1.1301x; 1.1301x over previous
import functools

import jax
import jax.numpy as jnp
from jax import lax
from jax.experimental import pallas as pl
from jax.experimental.pallas import tpu as pltpu
from jax.experimental.pallas import tpu_sc as plsc

B = 16384
D = 64
N = 1000000
SR = 120
STAGE = 128
NB = B // STAGE


@functools.cache
def _build_kernel():
    info = plsc.get_sparse_core_info()
    nw = info.num_cores * info.num_subcores
    per = N // nw
    mesh = plsc.VectorSubcoreMesh(core_axis_name="c", subcore_axis_name="s")
    i32 = jnp.int32

    @functools.partial(
        pl.kernel,
        mesh=mesh,
        out_type=(
            jax.ShapeDtypeStruct((B, 2 * D), jnp.float32),
            jax.ShapeDtypeStruct((B, 2 * D), jnp.float32),
        ),
        scratch_types=[
            pltpu.VMEM((B,), i32),
            pltpu.VMEM((B + 16,), i32),
            pltpu.VMEM((B + 16,), i32),
            pltpu.VMEM((SR, D), jnp.float32),
            pltpu.VMEM((SR, D), jnp.float32),
            pltpu.VMEM((STAGE, 2 * D), jnp.float32),
            pltpu.VMEM((STAGE, 2 * D), jnp.float32),
            pltpu.VMEM((NB, STAGE), i32),
            pltpu.VMEM((16,), i32),
            pltpu.VMEM((16,), i32),
            pltpu.SemaphoreType.DMA,
            pltpu.SemaphoreType.DMA,
        ],
        compiler_params=pltpu.CompilerParams(needs_layout_passes=False),
    )
    def k(ids_hbm, ws_hbm, wa_hbm, out_s_hbm, out_a_hbm,
          idx_v, sel_id, sel_pos, slab_s, slab_a, stage_s, stage_a,
          poslist, hit_id, hit_pos, sem_s, sem_a):
        wid = lax.axis_index("s") * info.num_cores + lax.axis_index("c")
        lo = wid * per // 8 * 8
        hi = (wid + 1) * per // 8 * 8
        iota = lax.iota(i32, 16)
        lane0 = iota == 0

        pltpu.sync_copy(ids_hbm, idx_v)

        def sel_body(g, off):
            v = idx_v[pl.ds(g * 16, 16)]
            m = (v >= lo) & (v < hi)
            plsc.store_compressed(sel_id.at[pl.ds(off, 16)], v, mask=m)
            plsc.store_compressed(sel_pos.at[pl.ds(off, 16)], iota + g * 16, mask=m)
            return off + plsc.all_reduce_population_count(m)[0]

        nsel = lax.fori_loop(0, B // 16, sel_body, jnp.int32(0))

        def flush(bb):
            cp_s = pltpu.async_copy(
                stage_s, out_s_hbm.at[poslist.at[bb]], sem_s)
            cp_a = pltpu.async_copy(
                stage_a, out_a_hbm.at[poslist.at[bb]], sem_a)
            cp_s.wait()
            cp_a.wait()

        n_sub = (hi - lo + SR - 1) // SR

        def sub_body(s, wrbb):
            wr, bb = wrbb
            sub_lo = lo + s * SR
            sub_hi = jnp.minimum(sub_lo + SR, hi)
            start = jnp.minimum(sub_lo, N - SR)
            cps = pltpu.async_copy(ws_hbm.at[pl.ds(start, SR)], slab_s, sem_s)
            cpa = pltpu.async_copy(wa_hbm.at[pl.ds(start, SR)], slab_a, sem_a)
            cps.wait()
            cpa.wait()

            def grp_body(g, wrbb):
                wr, bb = wrbb
                v = sel_id[pl.ds(g * 16, 16)]
                valid = (g * 16 + iota) < nsel
                m = valid & (v >= sub_lo) & (v < sub_hi)
                c = plsc.all_reduce_population_count(m)[0]
                plsc.store_compressed(hit_id.at[pl.ds(0, 16)], v, mask=m)
                plsc.store_compressed(
                    hit_pos.at[pl.ds(0, 16)], sel_pos[pl.ds(g * 16, 16)], mask=m)

                def one_hit(h, wrbb):
                    wr, bb = wrbb
                    hsp = jnp.full((16,), 0, i32) + h
                    rr = plsc.load_gather(hit_id, [hsp])[0] - start
                    pv = plsc.load_gather(hit_pos, [hsp])
                    for q in range(D // 16):
                        cs = pl.ds(q * 16, 16)
                        stage_s[wr, cs] = slab_s[rr, cs]
                        stage_a[wr, cs] = slab_a[rr, cs]
                    plsc.store_scatter(
                        poslist,
                        [jnp.full((16,), 0, i32) + bb,
                         jnp.full((16,), 0, i32) + wr],
                        pv, mask=lane0)

                    @pl.when(wr + 1 == STAGE)
                    def _():
                        flush(bb)

                    full = wr + 1 == STAGE
                    return (jnp.where(full, 0, wr + 1).astype(i32),
                            jnp.where(full, bb + 1, bb).astype(i32))

                return lax.fori_loop(0, c, one_hit, (wr, bb))

            return lax.fori_loop(0, (nsel + 15) // 16, grp_body, (wr, bb))

        wr, bb = lax.fori_loop(0, n_sub, sub_body,
                               (jnp.int32(0), jnp.int32(0)))

        @pl.when(wr > 0)
        def _():
            lastv = jnp.full((16,), 0, i32) + (wr - 1)
            bsp = jnp.full((16,), 0, i32) + bb
            pv = plsc.load_gather(poslist, [bsp, lastv])

            def pad_body(p, _):
                for q in range(D // 16):
                    cs = pl.ds(q * 16, 16)
                    stage_s[p, cs] = stage_s[wr - 1, cs]
                    stage_a[p, cs] = stage_a[wr - 1, cs]
                plsc.store_scatter(
                    poslist, [bsp, jnp.full((16,), 0, i32) + p], pv, mask=lane0)
                return 0

            lax.fori_loop(wr, STAGE, pad_body, 0)
            flush(bb)

    return k


def kernel(instance_ids, W_shape, W_appearance):
    ids = instance_ids.astype(jnp.int32)
    out_s, out_a = _build_kernel()(ids, W_shape, W_appearance)
    return (out_s[:, :D], out_a[:, :D])


# speedup vs baseline: 2.2522x; 1.9928x over previous
import functools

import jax
import jax.numpy as jnp
from jax import lax
from jax.experimental import pallas as pl
from jax.experimental.pallas import tpu as pltpu
from jax.experimental.pallas import tpu_sc as plsc

B = 16384
D = 64
CH = 128


@functools.cache
def _build_kernel():
    info = plsc.get_sparse_core_info()
    nw = info.num_cores * info.num_subcores
    b_per_w = B // nw
    n_ch = b_per_w // CH
    mesh = plsc.VectorSubcoreMesh(core_axis_name="c", subcore_axis_name="s")

    @functools.partial(
        pl.kernel,
        mesh=mesh,
        out_type=(
            jax.ShapeDtypeStruct((B, D), jnp.float32),
            jax.ShapeDtypeStruct((B, D), jnp.float32),
        ),
        scratch_types=[
            pltpu.VMEM((b_per_w,), jnp.int32),
            [pltpu.VMEM((CH, D), jnp.float32)] * 2,
            [pltpu.VMEM((CH, D), jnp.float32)] * 2,
            [pltpu.SemaphoreType.DMA] * 2,
            [pltpu.SemaphoreType.DMA] * 2,
            [pltpu.SemaphoreType.DMA] * 2,
        ],
        compiler_params=pltpu.CompilerParams(needs_layout_passes=False),
    )
    def k(ids_hbm, ws_hbm, wa_hbm, out_s_hbm, out_a_hbm,
          idx_v, rows_s, rows_a, gsems, osems_s, osems_a):
        wid = lax.axis_index("s") * info.num_cores + lax.axis_index("c")
        base = wid * b_per_w
        pltpu.sync_copy(ids_hbm.at[pl.ds(base, b_per_w)], idx_v)

        def fire(ch):
            bi = ch % 2

            def grp(g):
                v = idx_v[pl.ds(ch * CH + g * 16, 16)]
                for l in range(16):
                    r = v[l]
                    i = g * 16 + l
                    pltpu.async_copy(
                        ws_hbm.at[pl.ds(r, 1)],
                        rows_s[bi].at[pl.ds(i, 1)], gsems[bi])
                    pltpu.async_copy(
                        wa_hbm.at[pl.ds(r, 1)],
                        rows_a[bi].at[pl.ds(i, 1)], gsems[bi])

            plsc.parallel_loop(0, CH // 16, 1, unroll=4)(grp)

        def drain_and_writeback(ch):
            bi = ch % 2
            pltpu.make_async_copy(
                ws_hbm.at[pl.ds(0, CH)], rows_s[bi], gsems[bi]).wait()
            pltpu.make_async_copy(
                wa_hbm.at[pl.ds(0, CH)], rows_a[bi], gsems[bi]).wait()
            pltpu.async_copy(
                rows_s[bi], out_s_hbm.at[pl.ds(base + ch * CH, CH)],
                osems_s[bi])
            pltpu.async_copy(
                rows_a[bi], out_a_hbm.at[pl.ds(base + ch * CH, CH)],
                osems_a[bi])

        def wait_out(ch):
            bi = ch % 2
            pltpu.make_async_copy(
                rows_s[bi], out_s_hbm.at[pl.ds(0, CH)], osems_s[bi]).wait()
            pltpu.make_async_copy(
                rows_a[bi], out_a_hbm.at[pl.ds(0, CH)], osems_a[bi]).wait()

        fire(0)
        fire(1)
        for ch in range(n_ch):
            drain_and_writeback(ch)
            if ch + 2 < n_ch:
                wait_out(ch)
                fire(ch + 2)
        wait_out(n_ch - 2)
        wait_out(n_ch - 1)

    return k


def kernel(instance_ids, W_shape, W_appearance):
    ids = instance_ids.astype(jnp.int32)
    return _build_kernel()(ids, W_shape, W_appearance)
